# Initial kernel scaffold; baseline (speedup 1.0000x reference)
import jax
import jax.numpy as jnp
from jax.experimental import pallas as pl


def kernel(x, params, adj_rows, adj_cols, triu_rows, triu_cols, deg):
    raise NotImplementedError("write your pallas kernel here")


# speedup vs baseline: 2.1188x; 2.1188x over previous
import functools

import jax
import jax.numpy as jnp
from jax import lax
from jax.experimental import pallas as pl
from jax.experimental.pallas import tpu as pltpu
from jax.experimental.pallas import tpu_sc as plsc

NC = 2
NS = 16
FC = 128
CE = 128


def _cdiv(a, b):
  return (a + b - 1) // b


@functools.lru_cache(maxsize=None)
def _spmm_chain(n_pad, n_chunks, e_pad):
  n = n_pad
  ept = e_pad // NS
  n_e = ept // CE
  nr = n // NS
  rows_total = n_chunks * n + 8
  out_sd = jax.ShapeDtypeStruct((rows_total, FC), jnp.float32)
  mesh = plsc.VectorSubcoreMesh(
      core_axis_name="c", subcore_axis_name="s",
      num_cores=NC, num_subcores=NS)

  def body(xf, cols3, rows2, d1, d2, d3, d4,
           acc, idxc, idxr, gbuf, zbuf):
    c = lax.axis_index("c")
    s = lax.axis_index("s")

    @pl.loop(0, 64 * (FC // 16))
    def _zfill(i):
      r = i // (FC // 16)
      j = i % (FC // 16)
      zbuf[r, pl.ds(j * 16, 16)] = jnp.zeros((16,), jnp.float32)

    r0 = s * nr

    def zero_acc():
      off = 0
      while off < nr:
        l = min(64, nr - off)
        pltpu.sync_copy(zbuf.at[pl.ds(0, l)], acc.at[pl.ds(r0 + off, l)])
        off += l

    for rep in range(n_chunks // NC):
      chunk = c + NC * rep
      coff = chunk * n
      srcs = [xf, d1, d2, d3]
      dsts = [d1, d2, d3, d4]
      for hop in range(4):
        src = srcs[hop]
        dst = dsts[hop]
        zero_acc()
        pltpu.sync_copy(cols3.at[chunk, pl.ds(s * n_e, n_e)], idxc)
        pltpu.sync_copy(rows2.at[pl.ds(s * n_e, n_e)], idxr)
        plsc.subcore_barrier()

        @pl.loop(0, n_e)
        def _edges(i):
          pltpu.sync_copy(src.at[idxc.at[i]], gbuf)
          pltpu.sync_copy(gbuf, acc.at[idxr.at[i]], add=True)

        plsc.subcore_barrier()
        pltpu.sync_copy(acc.at[pl.ds(r0, nr)],
                        dst.at[pl.ds(coff + r0, nr)])

        @pl.when(s == 0)
        def _():
          pltpu.sync_copy(zbuf.at[pl.ds(0, 1)],
                          dst.at[pl.ds(n_chunks * n + c, 1)])

        plsc.subcore_barrier()

  kern = pl.kernel(
      body,
      out_type=(out_sd, out_sd, out_sd, out_sd),
      mesh=mesh,
      scratch_types=[
          pltpu.VMEM_SHARED((n, FC), jnp.float32),
          pltpu.VMEM((n_e, CE), jnp.int32),
          pltpu.VMEM((n_e, CE), jnp.int32),
          pltpu.VMEM((CE, FC), jnp.float32),
          pltpu.VMEM((64, FC), jnp.float32),
      ],
  )
  return kern


def _dense_body(x, a1, a2, a4, wx, w1, w2, w3, sc, sh, o, *, relu_cols):
  acc = jnp.dot(x[...], wx[...], preferred_element_type=jnp.float32)
  acc += jnp.dot(a1[...], w1[...], preferred_element_type=jnp.float32)
  acc += jnp.dot(a2[...], w2[...], preferred_element_type=jnp.float32)
  acc += jnp.dot(a4[...], w3[...], preferred_element_type=jnp.float32)
  col = lax.broadcasted_iota(jnp.int32, acc.shape, 1)
  y = jnp.where(col < relu_cols, jnp.maximum(acc, 0.0), acc)
  o[...] = y * sc[0:1, :] + sh[0:1, :]


def _dense_layer(x, a1, a2, a4, wx, w1, w2, w3, sc8, sh8, relu_cols):
  n, fin = x.shape
  fout2 = wx.shape[1]
  mb = 2000
  grid = (n // mb,)
  in_row = pl.BlockSpec((mb, fin), lambda m: (m, 0))
  w_spec = pl.BlockSpec((fin, fout2), lambda m: (0, 0))
  v_spec = pl.BlockSpec((8, fout2), lambda m: (0, 0))
  return pl.pallas_call(
      functools.partial(_dense_body, relu_cols=relu_cols),
      grid=grid,
      in_specs=[in_row, in_row, in_row, in_row,
                w_spec, w_spec, w_spec, w_spec, v_spec, v_spec],
      out_specs=pl.BlockSpec((mb, fout2), lambda m: (m, 0)),
      out_shape=jax.ShapeDtypeStruct((n, fout2), jnp.float32),
  )(x, a1, a2, a4, wx, w1, w2, w3, sc8, sh8)


def _z_body(h, w, b, o):
  o[...] = jnp.dot(h[...], w[...],
                   preferred_element_type=jnp.float32) + b[0:1, :]


def _z_matmul(h, w128, b8):
  n, fin = h.shape
  mb = 2000
  return pl.pallas_call(
      _z_body,
      grid=(n // mb,),
      in_specs=[pl.BlockSpec((mb, fin), lambda m: (m, 0)),
                pl.BlockSpec((fin, 128), lambda m: (0, 0)),
                pl.BlockSpec((8, 128), lambda m: (0, 0))],
      out_specs=pl.BlockSpec((mb, 128), lambda m: (m, 0)),
      out_shape=jax.ShapeDtypeStruct((n, 128), jnp.float32),
  )(h, w128, b8)


@functools.lru_cache(maxsize=None)
def _edge_readout(n, m_pad):
  mpt = m_pad // (NC * NS)
  n_c = mpt // CE
  mesh = plsc.VectorSubcoreMesh(
      core_axis_name="c", subcore_axis_name="s",
      num_cores=NC, num_subcores=NS)

  def body(z, lo2, hi2, out, idxl, idxh, bufa, bufb, bufo):
    c = lax.axis_index("c")
    s = lax.axis_index("s")
    w = s * NC + c
    base = w * n_c

    @pl.loop(0, n_c)
    def _chunk(i):
      pltpu.sync_copy(lo2.at[base + i], idxl)
      pltpu.sync_copy(hi2.at[base + i], idxh)
      pltpu.sync_copy(z.at[idxl], bufa)
      pltpu.sync_copy(z.at[idxh], bufb)

      @pl.loop(0, CE)
      def _add(r):
        bufo[r, pl.ds(0, 16)] = bufa[r, pl.ds(0, 16)] + bufb[r, pl.ds(0, 16)]

      pltpu.sync_copy(bufo, out.at[pl.ds((base + i) * CE, CE)])

  return pl.kernel(
      body,
      out_type=jax.ShapeDtypeStruct((m_pad, 128), jnp.float32),
      mesh=mesh,
      scratch_types=[
          pltpu.VMEM((CE,), jnp.int32),
          pltpu.VMEM((CE,), jnp.int32),
          pltpu.VMEM((CE, 128), jnp.float32),
          pltpu.VMEM((CE, 128), jnp.float32),
          pltpu.VMEM((CE, 128), jnp.float32),
      ],
  )


EPS = 1e-3


def _bn_scale_shift(p):
  s = p["gamma"] / jnp.sqrt(p["var"] + EPS)
  t = p["beta"] - p["mean"] * s
  return s, t


def kernel(x, params, adj_rows, adj_cols, triu_rows, triu_cols, deg):
  del deg
  n, f0 = x.shape
  e = adj_rows.shape[0]
  m = triu_rows.shape[0]

  grp = NS * CE * 8
  e_pad = _cdiv(e, grp) * grp
  pad = e_pad - e
  rows_p = jnp.concatenate(
      [adj_rows.astype(jnp.int32), jnp.zeros((pad,), jnp.int32)])
  rows2 = rows_p.reshape(e_pad // CE, CE)
  cols_p = adj_cols.astype(jnp.int32)

  n_pad = _cdiv(n, NS * 8) * NS * 8

  def make_cols3(n_chunks):
    offs = jnp.arange(n_chunks, dtype=jnp.int32) * n_pad
    main = cols_p[None, :] + offs[:, None]
    zrow = (jnp.full((n_chunks, pad), n_chunks * n_pad, jnp.int32)
            + (jnp.arange(n_chunks, dtype=jnp.int32) % NC)[:, None])
    return jnp.concatenate([main, zrow], axis=1).reshape(
        n_chunks, e_pad // CE, CE)

  def chunk_major(h, n_chunks):
    hf = h.reshape(n, n_chunks, FC).transpose(1, 0, 2)
    hf = jnp.pad(hf, ((0, 0), (0, n_pad - n), (0, 0))).reshape(
        n_chunks * n_pad, FC)
    return jnp.concatenate([hf, jnp.zeros((8, FC), jnp.float32)], axis=0)

  def unchunk(af, n_chunks, fin):
    return af[:n_chunks * n_pad].reshape(n_chunks, n_pad, FC)[
        :, :n, :].transpose(1, 0, 2).reshape(n, fin)

  h = x
  for layer in params["layers"]:
    fin = h.shape[1]
    n_chunks = fin // FC
    xf = chunk_major(h, n_chunks)
    cols3 = make_cols3(n_chunks)
    a1f, a2f, _a3f, a4f = _spmm_chain(n_pad, n_chunks, e_pad)(
        xf, cols3, rows2)
    a1 = unchunk(a1f, n_chunks, fin)
    a2 = unchunk(a2f, n_chunks, fin)
    a4 = unchunk(a4f, n_chunks, fin)

    wxa = layer["alpha"][0] + layer["alpha"][1] + layer["alpha"][2]
    wxb = layer["beta"][0] + layer["beta"][1] + layer["beta"][2]
    wx = jnp.concatenate([wxa, wxb], axis=1)
    ws = [jnp.concatenate([layer["alpha"][3 + i], layer["beta"][3 + i]],
                          axis=1) for i in range(3)]
    sa, ta = _bn_scale_shift(layer["bn_alpha"])
    sb, tb = _bn_scale_shift(layer["bn_beta"])
    fout = sa.shape[0]
    sc8 = jnp.tile(jnp.concatenate([sa, sb])[None, :], (8, 1))
    sh8 = jnp.tile(jnp.concatenate([ta, tb])[None, :], (8, 1))
    h = _dense_layer(h, a1, a2, a4, wx, ws[0], ws[1], ws[2],
                     sc8, sh8, relu_cols=fout)

  wz = params["edge_W"]
  nm = wz.shape[1]
  wz128 = jnp.zeros((wz.shape[0], 128), jnp.float32).at[:, :nm].set(wz)
  bz8 = jnp.tile(
      jnp.zeros((128,), jnp.float32).at[:nm].set(params["edge_b"])[None, :],
      (8, 1))
  z = _z_matmul(h, wz128, bz8)

  mg = NC * NS * CE
  m_pad = _cdiv(m, mg) * mg
  lo2 = jnp.concatenate(
      [triu_rows.astype(jnp.int32),
       jnp.zeros((m_pad - m,), jnp.int32)]).reshape(m_pad // CE, CE)
  hi2 = jnp.concatenate(
      [triu_cols.astype(jnp.int32),
       jnp.zeros((m_pad - m,), jnp.int32)]).reshape(m_pad // CE, CE)
  out = _edge_readout(n, m_pad)(z, lo2, hi2)
  return out[:m, :nm]


# speedup vs baseline: 2.6687x; 1.2595x over previous
import functools

import jax
import jax.numpy as jnp
from jax import lax
from jax.experimental import pallas as pl
from jax.experimental.pallas import tpu as pltpu
from jax.experimental.pallas import tpu_sc as plsc

NC = 2
NS = 16
FC = 128
CE = 128
NBUF = 2
NSLOT = 8
PRE = 4


def _cdiv(a, b):
  return (a + b - 1) // b


@functools.lru_cache(maxsize=None)
def _spmm_chain(n_pad, n_chunks, e_pad):
  n = n_pad
  ept = e_pad // NS
  n_e = ept // CE
  nr = n // NS
  rows_total = n_chunks * n + 8
  out_sd = jax.ShapeDtypeStruct((rows_total, FC), jnp.float32)
  mesh = plsc.VectorSubcoreMesh(
      core_axis_name="c", subcore_axis_name="s",
      num_cores=NC, num_subcores=NS)

  def body(xf, cols3, rows2, d1, d2, d3, d4,
           acc, idxc, idxr, gbuf, zbuf, sem_ic, sem_ir, sem_g, sem_s):
    c = lax.axis_index("c")
    s = lax.axis_index("s")

    @pl.loop(0, 64 * (FC // 16))
    def _zfill(i):
      r = i // (FC // 16)
      j = i % (FC // 16)
      zbuf[r, pl.ds(j * 16, 16)] = jnp.zeros((16,), jnp.float32)

    r0 = s * nr

    def zero_acc():
      off = 0
      while off < nr:
        l = min(64, nr - off)
        pltpu.sync_copy(zbuf.at[pl.ds(0, l)], acc.at[pl.ds(r0 + off, l)])
        off += l

    for rep in range(n_chunks // NC):
      chunk = c + NC * rep
      coff = chunk * n
      srcs = [xf, d1, d2, d3]
      dsts = [d1, d2, d3, d4]
      base_e = s * n_e
      for hop in range(4):
        src = srcs[hop]
        dst = dsts[hop]
        zero_acc()
        plsc.subcore_barrier()

        def ic_d(i, q):
          return pltpu.make_async_copy(
              cols3.at[chunk, base_e + i], idxc.at[q], sem_ic.at[q])

        def ir_d(i, q):
          return pltpu.make_async_copy(
              rows2.at[base_e + i], idxr.at[q], sem_ir.at[q])

        def g_d(i, k):
          return pltpu.make_async_copy(
              src.at[idxc.at[k % NSLOT]], gbuf.at[k % NBUF],
              sem_g.at[k % NBUF])

        def s_d(i, k):
          return pltpu.make_async_copy(
              gbuf.at[k % NBUF], acc.at[idxr.at[k % NSLOT]],
              sem_s.at[k % NBUF])

        def step(i, k, prefetch, first, gnext):
          if prefetch:
            ic_d(i + PRE, (k + PRE) % NSLOT).start()
            ir_d(i + PRE, (k + PRE) % NSLOT).start()
          if not first:
            s_d(i - 1, k - 1).wait()
          if gnext:
            ic_d(i + 1, (k + 1) % NSLOT).wait()
            g_d(i + 1, k + 1).start()
          g_d(i, k).wait()
          ir_d(i, k % NSLOT).wait()
          s_d(i, k).start(add=True)

        for q in range(PRE):
          ic_d(q, q).start()
          ir_d(q, q).start()
        ic_d(0, 0).wait()
        g_d(0, 0).start()
        for i in range(NSLOT):
          step(i, i, prefetch=True, first=(i == 0), gnext=True)

        @pl.loop(1, n_e // NSLOT - 1)
        def _edges(g):
          for k in range(NSLOT):
            step(g * NSLOT + k, k, prefetch=True, first=False, gnext=True)

        for k in range(NSLOT):
          i = n_e - NSLOT + k
          step(i, k, prefetch=(i + PRE < n_e), first=False,
               gnext=(i + 1 < n_e))
        s_d(n_e - 1, n_e - 1).wait()

        plsc.subcore_barrier()
        pltpu.sync_copy(acc.at[pl.ds(r0, nr)],
                        dst.at[pl.ds(coff + r0, nr)])

        @pl.when(s == 0)
        def _():
          pltpu.sync_copy(zbuf.at[pl.ds(0, 1)],
                          dst.at[pl.ds(n_chunks * n + c, 1)])

        plsc.subcore_barrier()

  kern = pl.kernel(
      body,
      out_type=(out_sd, out_sd, out_sd, out_sd),
      mesh=mesh,
      scratch_types=[
          pltpu.VMEM_SHARED((n, FC), jnp.float32),
          pltpu.VMEM((NSLOT, CE), jnp.int32),
          pltpu.VMEM((NSLOT, CE), jnp.int32),
          pltpu.VMEM((NBUF, CE, FC), jnp.float32),
          pltpu.VMEM((64, FC), jnp.float32),
          pltpu.SemaphoreType.DMA((NSLOT,)),
          pltpu.SemaphoreType.DMA((NSLOT,)),
          pltpu.SemaphoreType.DMA((NBUF,)),
          pltpu.SemaphoreType.DMA((NBUF,)),
      ],
  )
  return kern


def _dense_body(x, a1, a2, a4, wx, w1, w2, w3, sc, sh, o, *, relu_cols):
  acc = jnp.dot(x[...], wx[...], preferred_element_type=jnp.float32)
  acc += jnp.dot(a1[...], w1[...], preferred_element_type=jnp.float32)
  acc += jnp.dot(a2[...], w2[...], preferred_element_type=jnp.float32)
  acc += jnp.dot(a4[...], w3[...], preferred_element_type=jnp.float32)
  col = lax.broadcasted_iota(jnp.int32, acc.shape, 1)
  y = jnp.where(col < relu_cols, jnp.maximum(acc, 0.0), acc)
  o[...] = y * sc[0:1, :] + sh[0:1, :]


def _dense_layer(x, a1, a2, a4, wx, w1, w2, w3, sc8, sh8, relu_cols):
  n, fin = x.shape
  fout2 = wx.shape[1]
  mb = 2000
  grid = (n // mb,)
  in_row = pl.BlockSpec((mb, fin), lambda m: (m, 0))
  w_spec = pl.BlockSpec((fin, fout2), lambda m: (0, 0))
  v_spec = pl.BlockSpec((8, fout2), lambda m: (0, 0))
  return pl.pallas_call(
      functools.partial(_dense_body, relu_cols=relu_cols),
      grid=grid,
      in_specs=[in_row, in_row, in_row, in_row,
                w_spec, w_spec, w_spec, w_spec, v_spec, v_spec],
      out_specs=pl.BlockSpec((mb, fout2), lambda m: (m, 0)),
      out_shape=jax.ShapeDtypeStruct((n, fout2), jnp.float32),
  )(x, a1, a2, a4, wx, w1, w2, w3, sc8, sh8)


def _z_body(h, w, b, o):
  o[...] = jnp.dot(h[...], w[...],
                   preferred_element_type=jnp.float32) + b[0:1, :]


def _z_matmul(h, w128, b8):
  n, fin = h.shape
  mb = 2000
  return pl.pallas_call(
      _z_body,
      grid=(n // mb,),
      in_specs=[pl.BlockSpec((mb, fin), lambda m: (m, 0)),
                pl.BlockSpec((fin, 128), lambda m: (0, 0)),
                pl.BlockSpec((8, 128), lambda m: (0, 0))],
      out_specs=pl.BlockSpec((mb, 128), lambda m: (m, 0)),
      out_shape=jax.ShapeDtypeStruct((n, 128), jnp.float32),
  )(h, w128, b8)


@functools.lru_cache(maxsize=None)
def _edge_readout(n, m_pad):
  mpt = m_pad // (NC * NS)
  n_c = mpt // CE
  mesh = plsc.VectorSubcoreMesh(
      core_axis_name="c", subcore_axis_name="s",
      num_cores=NC, num_subcores=NS)

  def body(z, lo2, hi2, out, idxl, idxh, bufa, bufb, bufo):
    c = lax.axis_index("c")
    s = lax.axis_index("s")
    w = s * NC + c
    base = w * n_c

    @pl.loop(0, n_c)
    def _chunk(i):
      pltpu.sync_copy(lo2.at[base + i], idxl)
      pltpu.sync_copy(hi2.at[base + i], idxh)
      pltpu.sync_copy(z.at[idxl], bufa)
      pltpu.sync_copy(z.at[idxh], bufb)

      @pl.loop(0, CE)
      def _add(r):
        bufo[r, pl.ds(0, 16)] = bufa[r, pl.ds(0, 16)] + bufb[r, pl.ds(0, 16)]

      pltpu.sync_copy(bufo, out.at[pl.ds((base + i) * CE, CE)])

  return pl.kernel(
      body,
      out_type=jax.ShapeDtypeStruct((m_pad, 128), jnp.float32),
      mesh=mesh,
      scratch_types=[
          pltpu.VMEM((CE,), jnp.int32),
          pltpu.VMEM((CE,), jnp.int32),
          pltpu.VMEM((CE, 128), jnp.float32),
          pltpu.VMEM((CE, 128), jnp.float32),
          pltpu.VMEM((CE, 128), jnp.float32),
      ],
  )


EPS = 1e-3


def _bn_scale_shift(p):
  s = p["gamma"] / jnp.sqrt(p["var"] + EPS)
  t = p["beta"] - p["mean"] * s
  return s, t


def kernel(x, params, adj_rows, adj_cols, triu_rows, triu_cols, deg):
  del deg
  n, f0 = x.shape
  e = adj_rows.shape[0]
  m = triu_rows.shape[0]

  grp = NS * CE * 8
  e_pad = _cdiv(e, grp) * grp
  pad = e_pad - e
  rows_p = jnp.concatenate(
      [adj_rows.astype(jnp.int32), jnp.zeros((pad,), jnp.int32)])
  rows2 = rows_p.reshape(e_pad // CE, CE)
  cols_p = adj_cols.astype(jnp.int32)

  n_pad = _cdiv(n, NS * 8) * NS * 8

  def make_cols3(n_chunks):
    offs = jnp.arange(n_chunks, dtype=jnp.int32) * n_pad
    main = cols_p[None, :] + offs[:, None]
    zrow = (jnp.full((n_chunks, pad), n_chunks * n_pad, jnp.int32)
            + (jnp.arange(n_chunks, dtype=jnp.int32) % NC)[:, None])
    return jnp.concatenate([main, zrow], axis=1).reshape(
        n_chunks, e_pad // CE, CE)

  def chunk_major(h, n_chunks):
    hf = h.reshape(n, n_chunks, FC).transpose(1, 0, 2)
    hf = jnp.pad(hf, ((0, 0), (0, n_pad - n), (0, 0))).reshape(
        n_chunks * n_pad, FC)
    return jnp.concatenate([hf, jnp.zeros((8, FC), jnp.float32)], axis=0)

  def unchunk(af, n_chunks, fin):
    return af[:n_chunks * n_pad].reshape(n_chunks, n_pad, FC)[
        :, :n, :].transpose(1, 0, 2).reshape(n, fin)

  h = x
  for layer in params["layers"]:
    fin = h.shape[1]
    n_chunks = fin // FC
    xf = chunk_major(h, n_chunks)
    cols3 = make_cols3(n_chunks)
    a1f, a2f, _a3f, a4f = _spmm_chain(n_pad, n_chunks, e_pad)(
        xf, cols3, rows2)
    a1 = unchunk(a1f, n_chunks, fin)
    a2 = unchunk(a2f, n_chunks, fin)
    a4 = unchunk(a4f, n_chunks, fin)

    wxa = layer["alpha"][0] + layer["alpha"][1] + layer["alpha"][2]
    wxb = layer["beta"][0] + layer["beta"][1] + layer["beta"][2]
    wx = jnp.concatenate([wxa, wxb], axis=1)
    ws = [jnp.concatenate([layer["alpha"][3 + i], layer["beta"][3 + i]],
                          axis=1) for i in range(3)]
    sa, ta = _bn_scale_shift(layer["bn_alpha"])
    sb, tb = _bn_scale_shift(layer["bn_beta"])
    fout = sa.shape[0]
    sc8 = jnp.tile(jnp.concatenate([sa, sb])[None, :], (8, 1))
    sh8 = jnp.tile(jnp.concatenate([ta, tb])[None, :], (8, 1))
    h = _dense_layer(h, a1, a2, a4, wx, ws[0], ws[1], ws[2],
                     sc8, sh8, relu_cols=fout)

  wz = params["edge_W"]
  nm = wz.shape[1]
  wz128 = jnp.zeros((wz.shape[0], 128), jnp.float32).at[:, :nm].set(wz)
  bz8 = jnp.tile(
      jnp.zeros((128,), jnp.float32).at[:nm].set(params["edge_b"])[None, :],
      (8, 1))
  z = _z_matmul(h, wz128, bz8)

  mg = NC * NS * CE
  m_pad = _cdiv(m, mg) * mg
  lo2 = jnp.concatenate(
      [triu_rows.astype(jnp.int32),
       jnp.zeros((m_pad - m,), jnp.int32)]).reshape(m_pad // CE, CE)
  hi2 = jnp.concatenate(
      [triu_cols.astype(jnp.int32),
       jnp.zeros((m_pad - m,), jnp.int32)]).reshape(m_pad // CE, CE)
  out = _edge_readout(n, m_pad)(z, lo2, hi2)
  return out[:m, :nm]


# speedup vs baseline: 2.8825x; 1.0801x over previous
import functools

import jax
import jax.numpy as jnp
from jax import lax
from jax.experimental import pallas as pl
from jax.experimental.pallas import tpu as pltpu
from jax.experimental.pallas import tpu_sc as plsc

NC = 2
NS = 16
FC = 128
CE = 128
NBUF = 2
NSLOT = 8
PRE = 4
MB = 2048


def _cdiv(a, b):
  return (a + b - 1) // b


@functools.lru_cache(maxsize=None)
def _spmm_chain(n_pad, n_chunks, e_pad):
  n = n_pad
  ept = e_pad // NS
  n_e = ept // CE
  nr = n // NS
  out_sd = jax.ShapeDtypeStruct((n_chunks * n, FC), jnp.float32)
  mesh = plsc.VectorSubcoreMesh(
      core_axis_name="c", subcore_axis_name="s",
      num_cores=NC, num_subcores=NS)

  def body(xf, cols3, rows2, d1, d2, d3, d4,
           acc, idxc, idxr, gbuf, zbuf, sem_ic, sem_ir, sem_g, sem_s):
    c = lax.axis_index("c")
    s = lax.axis_index("s")

    @pl.loop(0, 64 * (FC // 16))
    def _zfill(i):
      r = i // (FC // 16)
      j = i % (FC // 16)
      zbuf[r, pl.ds(j * 16, 16)] = jnp.zeros((16,), jnp.float32)

    r0 = s * nr

    def zero_acc():
      off = 0
      while off < nr:
        l = min(64, nr - off)
        pltpu.sync_copy(zbuf.at[pl.ds(0, l)], acc.at[pl.ds(r0 + off, l)])
        off += l

    for rep in range(n_chunks // NC):
      chunk = c + NC * rep
      coff = chunk * n
      srcs = [xf, d1, d2, d3]
      dsts = [d1, d2, d3, d4]
      base_e = s * n_e
      for hop in range(4):
        src = srcs[hop]
        dst = dsts[hop]
        zero_acc()
        plsc.subcore_barrier()

        def ic_d(i, q):
          return pltpu.make_async_copy(
              cols3.at[chunk, base_e + i], idxc.at[q], sem_ic.at[q])

        def ir_d(i, q):
          return pltpu.make_async_copy(
              rows2.at[base_e + i], idxr.at[q], sem_ir.at[q])

        def g_d(i, k):
          return pltpu.make_async_copy(
              src.at[idxc.at[k % NSLOT]], gbuf.at[k % NBUF],
              sem_g.at[k % NBUF])

        def s_d(i, k):
          return pltpu.make_async_copy(
              gbuf.at[k % NBUF], acc.at[idxr.at[k % NSLOT]],
              sem_s.at[k % NBUF])

        def step(i, k, prefetch, first, gnext):
          if prefetch:
            ic_d(i + PRE, (k + PRE) % NSLOT).start()
            ir_d(i + PRE, (k + PRE) % NSLOT).start()
          if not first:
            s_d(i - 1, k - 1).wait()
          if gnext:
            ic_d(i + 1, (k + 1) % NSLOT).wait()
            g_d(i + 1, k + 1).start()
          g_d(i, k).wait()
          ir_d(i, k % NSLOT).wait()
          s_d(i, k).start(add=True)

        for q in range(PRE):
          ic_d(q, q).start()
          ir_d(q, q).start()
        ic_d(0, 0).wait()
        g_d(0, 0).start()
        for i in range(NSLOT):
          step(i, i, prefetch=True, first=(i == 0), gnext=True)

        @pl.loop(1, n_e // NSLOT - 1)
        def _edges(g):
          for k in range(NSLOT):
            step(g * NSLOT + k, k, prefetch=True, first=False, gnext=True)

        for k in range(NSLOT):
          i = n_e - NSLOT + k
          step(i, k, prefetch=(i + PRE < n_e), first=False,
               gnext=(i + 1 < n_e))
        s_d(n_e - 1, n_e - 1).wait()

        plsc.subcore_barrier()
        pltpu.sync_copy(acc.at[pl.ds(r0, nr)],
                        dst.at[pl.ds(coff + r0, nr)])
        plsc.subcore_barrier()

  kern = pl.kernel(
      body,
      out_type=(out_sd, out_sd, out_sd, out_sd),
      mesh=mesh,
      scratch_types=[
          pltpu.VMEM_SHARED((n, FC), jnp.float32),
          pltpu.VMEM((NSLOT, CE), jnp.int32),
          pltpu.VMEM((NSLOT, CE), jnp.int32),
          pltpu.VMEM((NBUF, CE, FC), jnp.float32),
          pltpu.VMEM((64, FC), jnp.float32),
          pltpu.SemaphoreType.DMA((NSLOT,)),
          pltpu.SemaphoreType.DMA((NSLOT,)),
          pltpu.SemaphoreType.DMA((NBUF,)),
          pltpu.SemaphoreType.DMA((NBUF,)),
      ],
  )
  return kern


def _dense_core(x, a1, a2, a4, wx, w1, w2, w3, accr):
  cstep = pl.program_id(1)
  part = jnp.dot(x[...], wx[...], preferred_element_type=jnp.float32)
  part += jnp.dot(a1[...], w1[...], preferred_element_type=jnp.float32)
  part += jnp.dot(a2[...], w2[...], preferred_element_type=jnp.float32)
  part += jnp.dot(a4[...], w3[...], preferred_element_type=jnp.float32)

  @pl.when(cstep == 0)
  def _():
    accr[...] = part

  @pl.when(cstep > 0)
  def _():
    accr[...] += part


def _epilogue(accr, sc, sh, relu_cols):
  acc = accr[...]
  col = lax.broadcasted_iota(jnp.int32, acc.shape, 1)
  y = jnp.where(col < relu_cols, jnp.maximum(acc, 0.0), acc)
  return y * sc[0:1, :] + sh[0:1, :]


def _mid_body(x, a1, a2, a4, wx, w1, w2, w3, sc, sh, o, accr, *, relu_cols):
  _dense_core(x, a1, a2, a4, wx, w1, w2, w3, accr)

  @pl.when(pl.program_id(1) == pl.num_programs(1) - 1)
  def _():
    y = _epilogue(accr, sc, sh, relu_cols)
    for k in range(o.shape[0]):
      o[k] = y[:, k * FC:(k + 1) * FC]


def _fin_body(x, a1, a2, a4, wx, w1, w2, w3, sc, sh, wz, bz, o, accr, *,
              relu_cols):
  _dense_core(x, a1, a2, a4, wx, w1, w2, w3, accr)

  @pl.when(pl.program_id(1) == pl.num_programs(1) - 1)
  def _():
    y = _epilogue(accr, sc, sh, relu_cols)
    o[...] = jnp.dot(y, wz[...],
                     preferred_element_type=jnp.float32) + bz[0:1, :]


def _dense_layer(n_pad, nc, xf, a1f, a2f, a4f, wx, w1, w2, w3, sc8, sh8,
                 relu_cols, wz=None, bz=None):
  fout2 = wx.shape[1]
  nco = fout2 // FC
  grid = (n_pad // MB, nc)
  rb = pl.BlockSpec((MB, FC), lambda m, c: (c * (n_pad // MB) + m, 0))
  wb = pl.BlockSpec((FC, fout2), lambda m, c: (c, 0))
  vb = pl.BlockSpec((8, fout2), lambda m, c: (0, 0))
  args = [xf, a1f, a2f, a4f, wx, w1, w2, w3, sc8, sh8]
  in_specs = [rb, rb, rb, rb, wb, wb, wb, wb, vb, vb]
  if wz is None:
    body = functools.partial(_mid_body, relu_cols=relu_cols)
    out_specs = pl.BlockSpec((nco, MB, FC), lambda m, c: (0, m, 0))
    out_shape = jax.ShapeDtypeStruct((nco, n_pad, FC), jnp.float32)
  else:
    body = functools.partial(_fin_body, relu_cols=relu_cols)
    in_specs += [pl.BlockSpec((fout2, 128), lambda m, c: (0, 0)),
                 pl.BlockSpec((8, 128), lambda m, c: (0, 0))]
    args += [wz, bz]
    out_specs = pl.BlockSpec((MB, 128), lambda m, c: (m, 0))
    out_shape = jax.ShapeDtypeStruct((n_pad, 128), jnp.float32)
  return pl.pallas_call(
      body,
      grid=grid,
      in_specs=in_specs,
      out_specs=out_specs,
      out_shape=out_shape,
      scratch_shapes=[pltpu.VMEM((MB, fout2), jnp.float32)],
  )(*args)


@functools.lru_cache(maxsize=None)
def _edge_readout(m_pad):
  n_c = m_pad // (NC * NS) // CE
  mesh = plsc.VectorSubcoreMesh(
      core_axis_name="c", subcore_axis_name="s",
      num_cores=NC, num_subcores=NS)

  def body(z, lo2, hi2, out, idxl, idxh, bufa, bufb, bufo):
    c = lax.axis_index("c")
    s = lax.axis_index("s")
    w = s * NC + c
    base = w * n_c

    @pl.loop(0, n_c)
    def _chunk(i):
      pltpu.sync_copy(lo2.at[base + i], idxl)
      pltpu.sync_copy(hi2.at[base + i], idxh)
      pltpu.sync_copy(z.at[idxl], bufa)
      pltpu.sync_copy(z.at[idxh], bufb)

      @pl.loop(0, CE)
      def _add(r):
        bufo[r, pl.ds(0, 16)] = bufa[r, pl.ds(0, 16)] + bufb[r, pl.ds(0, 16)]

      pltpu.sync_copy(bufo, out.at[pl.ds((base + i) * CE, CE)])

  return pl.kernel(
      body,
      out_type=jax.ShapeDtypeStruct((m_pad, 128), jnp.float32),
      mesh=mesh,
      scratch_types=[
          pltpu.VMEM((CE,), jnp.int32),
          pltpu.VMEM((CE,), jnp.int32),
          pltpu.VMEM((CE, 128), jnp.float32),
          pltpu.VMEM((CE, 128), jnp.float32),
          pltpu.VMEM((CE, 128), jnp.float32),
      ],
  )


EPS = 1e-3


def _bn_scale_shift(p):
  s = p["gamma"] / jnp.sqrt(p["var"] + EPS)
  t = p["beta"] - p["mean"] * s
  return s, t


def kernel(x, params, adj_rows, adj_cols, triu_rows, triu_cols, deg):
  del deg
  n, f0 = x.shape
  e = adj_rows.shape[0]
  m = triu_rows.shape[0]
  n_pad = max(_cdiv(n + 1, NS * 8) * NS * 8, _cdiv(n, MB) * MB)
  assert n_pad % MB == 0 and n_pad % (NS * 8) == 0

  grp = NS * CE * NSLOT
  e_pad = _cdiv(e, grp) * grp
  pad = e_pad - e
  rows_p = jnp.concatenate(
      [adj_rows.astype(jnp.int32), jnp.full((pad,), n, jnp.int32)])
  rows2 = rows_p.reshape(e_pad // CE, CE)
  cols_p = adj_cols.astype(jnp.int32)

  def make_cols3(n_chunks):
    offs = jnp.arange(n_chunks, dtype=jnp.int32) * n_pad
    main = jnp.concatenate([cols_p, jnp.zeros((pad,), jnp.int32)])
    return (main[None, :] + offs[:, None]).reshape(
        n_chunks, e_pad // CE, CE)

  h = None
  nc = f0 // FC
  xf = x.reshape(n, nc, FC).transpose(1, 0, 2)
  xf = jnp.pad(xf, ((0, 0), (0, n_pad - n), (0, 0))).reshape(
      nc * n_pad, FC)

  n_layers = len(params["layers"])
  for li, layer in enumerate(params["layers"]):
    a1f, a2f, _a3f, a4f = _spmm_chain(n_pad, nc, e_pad)(
        xf, make_cols3(nc), rows2)

    wxa = layer["alpha"][0] + layer["alpha"][1] + layer["alpha"][2]
    wxb = layer["beta"][0] + layer["beta"][1] + layer["beta"][2]
    wx = jnp.concatenate([wxa, wxb], axis=1)
    ws = [jnp.concatenate([layer["alpha"][3 + i], layer["beta"][3 + i]],
                          axis=1) for i in range(3)]
    sa, ta = _bn_scale_shift(layer["bn_alpha"])
    sb, tb = _bn_scale_shift(layer["bn_beta"])
    fout = sa.shape[0]
    sc8 = jnp.tile(jnp.concatenate([sa, sb])[None, :], (8, 1))
    sh8 = jnp.tile(jnp.concatenate([ta, tb])[None, :], (8, 1))

    if li < n_layers - 1:
      hf = _dense_layer(n_pad, nc, xf, a1f, a2f, a4f, wx,
                        ws[0], ws[1], ws[2], sc8, sh8, relu_cols=fout)
      nc = hf.shape[0]
      xf = hf.reshape(nc * n_pad, FC)
    else:
      wz = params["edge_W"]
      nm = wz.shape[1]
      wz128 = jnp.zeros((wz.shape[0], 128), jnp.float32).at[:, :nm].set(wz)
      bz8 = jnp.tile(
          jnp.zeros((128,), jnp.float32).at[:nm].set(
              params["edge_b"])[None, :], (8, 1))
      z = _dense_layer(n_pad, nc, xf, a1f, a2f, a4f, wx,
                       ws[0], ws[1], ws[2], sc8, sh8, relu_cols=fout,
                       wz=wz128, bz=bz8)

  mg = NC * NS * CE
  m_pad = _cdiv(m, mg) * mg
  lo2 = jnp.concatenate(
      [triu_rows.astype(jnp.int32),
       jnp.zeros((m_pad - m,), jnp.int32)]).reshape(m_pad // CE, CE)
  hi2 = jnp.concatenate(
      [triu_cols.astype(jnp.int32),
       jnp.zeros((m_pad - m,), jnp.int32)]).reshape(m_pad // CE, CE)
  out = _edge_readout(m_pad)(z, lo2, hi2)
  return out[:m, :params["edge_W"].shape[1]]
